# baseline (speedup 1.0000x reference)
import jax
import jax.numpy as jnp
from jax.experimental import pallas as pl
from jax.experimental.pallas import tpu as pltpu

ROWS_PER_BLOCK = 8


def _body(ts_ref, acp_ref, x_ref, n_ref, o_ref):
    i = pl.program_id(0)
    base = i * ROWS_PER_BLOCK
    for r in range(ROWS_PER_BLOCK):
        t = ts_ref[base + r]
        a = acp_ref[t]
        sa = jnp.sqrt(a)
        sb = jnp.sqrt(1.0 - a)
        o_ref[r] = sa * x_ref[r] + sb * n_ref[r]


def kernel(original_samples, noise, timesteps, alphas_cumprod):
    b = original_samples.shape[0]
    x = original_samples.reshape(b, 128, 128)
    n = noise.reshape(b, 128, 128)
    ts = timesteps.astype(jnp.int32)

    grid = (b // ROWS_PER_BLOCK,)
    blk = pl.BlockSpec((ROWS_PER_BLOCK, 128, 128), lambda i: (i, 0, 0))
    out = pl.pallas_call(
        _body,
        grid=grid,
        in_specs=[
            pl.BlockSpec(memory_space=pltpu.SMEM),
            pl.BlockSpec(memory_space=pltpu.SMEM),
            blk,
            blk,
        ],
        out_specs=blk,
        out_shape=jax.ShapeDtypeStruct((b, 128, 128), jnp.float32),
        compiler_params=pltpu.CompilerParams(
            dimension_semantics=("arbitrary",),
        ),
    )(ts, alphas_cumprod, x, n)
    return out.reshape(original_samples.shape)
